# Initial kernel scaffold; baseline (speedup 1.0000x reference)
import jax
import jax.numpy as jnp
from jax.experimental import pallas as pl


def kernel(x, edge_index, edge_attr, batch, W0, a_src0, a_dst0, b0, g0, be0, W1, a_src1, a_dst1, b1, g1, be1, lin1_W, lin1_b, gf, bf, lin2_W, lin2_b):
    raise NotImplementedError("write your pallas kernel here")


# speedup vs baseline: 67.9342x; 67.9342x over previous
import functools

import jax
import jax.numpy as jnp
from jax import lax
from jax.experimental import pallas as pl
from jax.experimental.pallas import tpu as pltpu
from jax.experimental.pallas import tpu_sc as plsc

N_ = 10000
E_ = 320000
F_ = 128
H_ = 8
C_ = 16
HC_ = 128
G_ = 64
TW_ = HC_ + 2 * H_

NBLK_ = 25
BLK_ = N_ // NBLK_

NWORK_ = 32
EW_ = E_ // NWORK_
KCH_ = 80
NCH_ = EW_ // KCH_
NPAD_ = 10240
ROWS_T_ = NPAD_ // 16


def _emit_tables(h, as_ref, ad_ref, tsrc_ref, tald_ref):
    als = jnp.dot(h, as_ref[...], preferred_element_type=jnp.float32)
    ald = jnp.dot(h, ad_ref[...], preferred_element_type=jnp.float32)
    z8 = jnp.zeros_like(als)
    tsrc_ref[...] = jnp.concatenate([h, als, z8], axis=1)
    tald_ref[...] = jnp.concatenate([ald, z8], axis=1)


def _stage_a_body(x_ref, w_ref, as_ref, ad_ref, tsrc_ref, tald_ref):
    h = jnp.dot(x_ref[...], w_ref[...], preferred_element_type=jnp.float32)
    _emit_tables(h, as_ref, ad_ref, tsrc_ref, tald_ref)


def _combine(p_ref, rrep_ref, b_ref, g_ref, be_ref):
    s = p_ref[0] + p_ref[1]
    num = s[:, :HC_]
    den8 = s[:, HC_:HC_ + H_]
    den = jnp.dot(den8, rrep_ref[...], preferred_element_type=jnp.float32)
    o = num / (den + 1e-16) + b_ref[...]
    m = jnp.mean(o, axis=1, keepdims=True)
    v = jnp.mean((o - m) ** 2, axis=1, keepdims=True)
    xn = (o - m) / jnp.sqrt(v + 1e-5) * g_ref[...] + be_ref[...]
    return jnp.maximum(xn, 0.0)


def _stage_b_body(p_ref, rrep_ref, b_ref, g_ref, be_ref, w_ref, as_ref,
                  ad_ref, tsrc_ref, tald_ref):
    xn = _combine(p_ref, rrep_ref, b_ref, g_ref, be_ref)
    h = jnp.dot(xn, w_ref[...], preferred_element_type=jnp.float32)
    _emit_tables(h, as_ref, ad_ref, tsrc_ref, tald_ref)


def _stage_c_body(p_ref, batch_ref, rrep_ref, b_ref, g_ref, be_ref,
                  l1w_ref, l1b_ref, gf_ref, bf_ref, l2w_ref, l2b_ref,
                  out_ref, pooled_scr, cnt_scr):
    i = pl.program_id(0)

    @pl.when(i == 0)
    def _():
        pooled_scr[...] = jnp.zeros_like(pooled_scr)
        cnt_scr[...] = jnp.zeros_like(cnt_scr)

    x2 = _combine(p_ref, rrep_ref, b_ref, g_ref, be_ref)
    bb = batch_ref[0]
    gid = lax.broadcasted_iota(jnp.int32, (G_, 1), 0)
    oh = (bb == gid).astype(jnp.float32)
    pooled_scr[...] += jnp.dot(oh, x2, preferred_element_type=jnp.float32)
    cnt_scr[...] += jnp.broadcast_to(
        jnp.sum(oh, axis=1, keepdims=True), (G_, HC_))

    @pl.when(i == NBLK_ - 1)
    def _():
        pm = pooled_scr[...] / jnp.maximum(cnt_scr[...], 1.0)
        a = jnp.dot(pm, l1w_ref[...], preferred_element_type=jnp.float32)
        a = a + l1b_ref[...]
        m = jnp.mean(a, axis=1, keepdims=True)
        v = jnp.mean((a - m) ** 2, axis=1, keepdims=True)
        a = (a - m) / jnp.sqrt(v + 1e-5) * gf_ref[...] + bf_ref[...]
        a = jnp.maximum(a, 0.0)
        out_ref[...] = (jnp.dot(a, l2w_ref[...],
                                preferred_element_type=jnp.float32)
                        + l2b_ref[...])


def _dense_stage_a(x2d, W, As_bd, Ad_bd):
    return pl.pallas_call(
        _stage_a_body,
        grid=(NBLK_,),
        in_specs=[
            pl.BlockSpec((BLK_, F_), lambda i: (i, 0)),
            pl.BlockSpec((F_, HC_), lambda i: (0, 0)),
            pl.BlockSpec((HC_, H_), lambda i: (0, 0)),
            pl.BlockSpec((HC_, H_), lambda i: (0, 0)),
        ],
        out_specs=[
            pl.BlockSpec((BLK_, TW_), lambda i: (i, 0)),
            pl.BlockSpec((BLK_, 2 * H_), lambda i: (i, 0)),
        ],
        out_shape=[
            jax.ShapeDtypeStruct((NPAD_, TW_), jnp.float32),
            jax.ShapeDtypeStruct((NPAD_, 2 * H_), jnp.float32),
        ],
    )(x2d, W, As_bd, Ad_bd)


def _dense_stage_b(parts, rrep, b, g, be, W, As_bd, Ad_bd):
    return pl.pallas_call(
        _stage_b_body,
        grid=(NBLK_,),
        in_specs=[
            pl.BlockSpec((2, BLK_, TW_), lambda i: (0, i, 0)),
            pl.BlockSpec((H_, HC_), lambda i: (0, 0)),
            pl.BlockSpec((1, HC_), lambda i: (0, 0)),
            pl.BlockSpec((1, HC_), lambda i: (0, 0)),
            pl.BlockSpec((1, HC_), lambda i: (0, 0)),
            pl.BlockSpec((HC_, HC_), lambda i: (0, 0)),
            pl.BlockSpec((HC_, H_), lambda i: (0, 0)),
            pl.BlockSpec((HC_, H_), lambda i: (0, 0)),
        ],
        out_specs=[
            pl.BlockSpec((BLK_, TW_), lambda i: (i, 0)),
            pl.BlockSpec((BLK_, 2 * H_), lambda i: (i, 0)),
        ],
        out_shape=[
            jax.ShapeDtypeStruct((NPAD_, TW_), jnp.float32),
            jax.ShapeDtypeStruct((NPAD_, 2 * H_), jnp.float32),
        ],
    )(parts, rrep, b, g, be, W, As_bd, Ad_bd)


def _dense_stage_c(parts, batch3, rrep, b, g, be, l1w, l1b, gf, bf, l2w, l2b):
    return pl.pallas_call(
        _stage_c_body,
        grid=(NBLK_,),
        in_specs=[
            pl.BlockSpec((2, BLK_, TW_), lambda i: (0, i, 0)),
            pl.BlockSpec((1, 1, BLK_), lambda i: (i, 0, 0)),
            pl.BlockSpec((H_, HC_), lambda i: (0, 0)),
            pl.BlockSpec((1, HC_), lambda i: (0, 0)),
            pl.BlockSpec((1, HC_), lambda i: (0, 0)),
            pl.BlockSpec((1, HC_), lambda i: (0, 0)),
            pl.BlockSpec((HC_, C_), lambda i: (0, 0)),
            pl.BlockSpec((1, C_), lambda i: (0, 0)),
            pl.BlockSpec((1, C_), lambda i: (0, 0)),
            pl.BlockSpec((1, C_), lambda i: (0, 0)),
            pl.BlockSpec((C_, C_), lambda i: (0, 0)),
            pl.BlockSpec((1, C_), lambda i: (0, 0)),
        ],
        out_specs=pl.BlockSpec((G_, C_), lambda i: (0, 0)),
        out_shape=jax.ShapeDtypeStruct((G_, C_), jnp.float32),
        scratch_shapes=[
            pltpu.VMEM((G_, HC_), jnp.float32),
            pltpu.VMEM((G_, HC_), jnp.float32),
        ],
    )(parts, batch3, rrep, b, g, be, l1w, l1b, gf, bf, l2w, l2b)


def _edge_kernel_body(tsrc_hbm, tald_hbm, ei_hbm, z_hbm, out_hbm,
                      src_v, dst_v, rows_v, ald_v, msg_v, acc, sem1, sem2):
    c = lax.axis_index("c")
    s = lax.axis_index("s")
    wid = c * 16 + s

    pltpu.sync_copy(z_hbm.at[pl.ds(s * ROWS_T_, ROWS_T_)],
                    acc.at[pl.ds(s * ROWS_T_, ROWS_T_)])
    plsc.subcore_barrier()

    base = wid * EW_

    def chunk(it, carry):
        off = base + it * KCH_
        pltpu.sync_copy(ei_hbm.at[pl.ds(off, KCH_)], src_v)
        pltpu.sync_copy(ei_hbm.at[pl.ds(E_ + off, KCH_)], dst_v)
        cp1 = pltpu.async_copy(tsrc_hbm.at[src_v], rows_v, sem1)
        cp2 = pltpu.async_copy(tald_hbm.at[dst_v], ald_v, sem2)
        cp1.wait()
        cp2.wait()

        def edge(i, carry2):
            als = rows_v[i, pl.ds(HC_, 16)]
            aldv = ald_v[i, :]
            e = als + aldv
            e = jnp.maximum(e, 0.2 * e)
            ex = jnp.exp(e)
            msg_v[i, pl.ds(HC_, 16)] = ex
            for h in range(H_):
                bc = ex[h]
                msg_v[i, pl.ds(h * 16, 16)] = (
                    rows_v[i, pl.ds(h * 16, 16)] * bc)
            return carry2

        lax.fori_loop(0, KCH_, edge, 0, unroll=False)
        pltpu.sync_copy(msg_v, acc.at[dst_v], add=True)
        return carry

    lax.fori_loop(0, NCH_, chunk, 0, unroll=False)
    plsc.subcore_barrier()
    pltpu.sync_copy(acc.at[pl.ds(s * ROWS_T_, ROWS_T_)],
                    out_hbm.at[c, pl.ds(s * ROWS_T_, ROWS_T_)])


def _edge_phase(tsrc, tald, edge_index, zeros_tab):
    mesh = plsc.VectorSubcoreMesh(core_axis_name="c", subcore_axis_name="s")
    fn = functools.partial(
        pl.kernel,
        mesh=mesh,
        compiler_params=pltpu.CompilerParams(use_tc_tiling_on_sc=False),
        out_type=jax.ShapeDtypeStruct((2, NPAD_, TW_), jnp.float32),
        scratch_types=[
            pltpu.VMEM((KCH_,), jnp.int32),
            pltpu.VMEM((KCH_,), jnp.int32),
            pltpu.VMEM((KCH_, TW_), jnp.float32),
            pltpu.VMEM((KCH_, 2 * H_), jnp.float32),
            pltpu.VMEM((KCH_, TW_), jnp.float32),
            pltpu.VMEM_SHARED((NPAD_, TW_), jnp.float32),
            pltpu.SemaphoreType.DMA,
            pltpu.SemaphoreType.DMA,
        ],
    )(_edge_kernel_body)
    return fn(tsrc, tald, edge_index, zeros_tab)


def _block_diag_attn(a):
    return (a[:, :, None] * jnp.eye(H_, dtype=jnp.float32)[:, None, :]
            ).reshape(H_ * C_, H_)


def kernel(x, edge_index, edge_attr, batch, W0, a_src0, a_dst0, b0, g0, be0,
           W1, a_src1, a_dst1, b1, g1, be1, lin1_W, lin1_b, gf, bf,
           lin2_W, lin2_b):
    x2d = jnp.squeeze(x.astype(jnp.float32), axis=1)
    ei = edge_index.astype(jnp.int32).reshape(2 * E_)
    batch3 = batch.astype(jnp.int32).reshape(NBLK_, 1, BLK_)

    As0 = _block_diag_attn(a_src0)
    Ad0 = _block_diag_attn(a_dst0)
    As1 = _block_diag_attn(a_src1)
    Ad1 = _block_diag_attn(a_dst1)
    rrep = (jnp.arange(HC_)[None, :] // C_ ==
            jnp.arange(H_)[:, None]).astype(jnp.float32)
    zeros_tab = jnp.zeros((NPAD_, TW_), jnp.float32)

    r2 = lambda t: t.reshape(1, -1).astype(jnp.float32)

    tsrc0, tald0 = _dense_stage_a(x2d, W0, As0, Ad0)
    parts0 = _edge_phase(tsrc0, tald0, ei, zeros_tab)
    tsrc1, tald1 = _dense_stage_b(parts0, rrep, r2(b0), r2(g0), r2(be0),
                                  W1, As1, Ad1)
    parts1 = _edge_phase(tsrc1, tald1, ei, zeros_tab)
    return _dense_stage_c(parts1, batch3, rrep, r2(b1), r2(g1), r2(be1),
                          lin1_W, r2(lin1_b), r2(gf), r2(bf),
                          lin2_W, r2(lin2_b))


# speedup vs baseline: 99.0056x; 1.4574x over previous
import functools

import jax
import jax.numpy as jnp
from jax import lax
from jax.experimental import pallas as pl
from jax.experimental.pallas import tpu as pltpu
from jax.experimental.pallas import tpu_sc as plsc

N_ = 10000
E_ = 320000
F_ = 128
H_ = 8
C_ = 16
HC_ = 128
G_ = 64
TW_ = HC_ + 2 * H_

NBLK_ = 25
BLK_ = N_ // NBLK_

NWORK_ = 32
EW_ = E_ // NWORK_
KCH_ = 80
NCH_ = EW_ // KCH_
NPAD_ = 10240
ROWS_T_ = NPAD_ // 16


def _emit_tables(h, as_ref, ad_ref, tsrc_ref, tald_ref):
    als = jnp.dot(h, as_ref[...], preferred_element_type=jnp.float32)
    ald = jnp.dot(h, ad_ref[...], preferred_element_type=jnp.float32)
    z8 = jnp.zeros_like(als)
    tsrc_ref[...] = jnp.concatenate([h, als, z8], axis=1)
    tald_ref[...] = jnp.concatenate([ald, z8], axis=1)


def _stage_a_body(x_ref, w_ref, as_ref, ad_ref, tsrc_ref, tald_ref):
    h = jnp.dot(x_ref[...], w_ref[...], preferred_element_type=jnp.float32)
    _emit_tables(h, as_ref, ad_ref, tsrc_ref, tald_ref)


def _combine(p_ref, rrep_ref, b_ref, g_ref, be_ref):
    s = p_ref[0] + p_ref[1]
    num = s[:, :HC_]
    den8 = s[:, HC_:HC_ + H_]
    den = jnp.dot(den8, rrep_ref[...], preferred_element_type=jnp.float32)
    o = num / (den + 1e-16) + b_ref[...]
    m = jnp.mean(o, axis=1, keepdims=True)
    v = jnp.mean((o - m) ** 2, axis=1, keepdims=True)
    xn = (o - m) / jnp.sqrt(v + 1e-5) * g_ref[...] + be_ref[...]
    return jnp.maximum(xn, 0.0)


def _stage_b_body(p_ref, rrep_ref, b_ref, g_ref, be_ref, w_ref, as_ref,
                  ad_ref, tsrc_ref, tald_ref):
    xn = _combine(p_ref, rrep_ref, b_ref, g_ref, be_ref)
    h = jnp.dot(xn, w_ref[...], preferred_element_type=jnp.float32)
    _emit_tables(h, as_ref, ad_ref, tsrc_ref, tald_ref)


def _stage_c_body(p_ref, batch_ref, rrep_ref, b_ref, g_ref, be_ref,
                  l1w_ref, l1b_ref, gf_ref, bf_ref, l2w_ref, l2b_ref,
                  out_ref, pooled_scr, cnt_scr):
    i = pl.program_id(0)

    @pl.when(i == 0)
    def _():
        pooled_scr[...] = jnp.zeros_like(pooled_scr)
        cnt_scr[...] = jnp.zeros_like(cnt_scr)

    x2 = _combine(p_ref, rrep_ref, b_ref, g_ref, be_ref)
    bb = batch_ref[0]
    gid = lax.broadcasted_iota(jnp.int32, (G_, 1), 0)
    oh = (bb == gid).astype(jnp.float32)
    pooled_scr[...] += jnp.dot(oh, x2, preferred_element_type=jnp.float32)
    cnt_scr[...] += jnp.broadcast_to(
        jnp.sum(oh, axis=1, keepdims=True), (G_, HC_))

    @pl.when(i == NBLK_ - 1)
    def _():
        pm = pooled_scr[...] / jnp.maximum(cnt_scr[...], 1.0)
        a = jnp.dot(pm, l1w_ref[...], preferred_element_type=jnp.float32)
        a = a + l1b_ref[...]
        m = jnp.mean(a, axis=1, keepdims=True)
        v = jnp.mean((a - m) ** 2, axis=1, keepdims=True)
        a = (a - m) / jnp.sqrt(v + 1e-5) * gf_ref[...] + bf_ref[...]
        a = jnp.maximum(a, 0.0)
        out_ref[...] = (jnp.dot(a, l2w_ref[...],
                                preferred_element_type=jnp.float32)
                        + l2b_ref[...])


def _dense_stage_a(x2d, W, As_bd, Ad_bd):
    return pl.pallas_call(
        _stage_a_body,
        grid=(NBLK_,),
        in_specs=[
            pl.BlockSpec((BLK_, F_), lambda i: (i, 0)),
            pl.BlockSpec((F_, HC_), lambda i: (0, 0)),
            pl.BlockSpec((HC_, H_), lambda i: (0, 0)),
            pl.BlockSpec((HC_, H_), lambda i: (0, 0)),
        ],
        out_specs=[
            pl.BlockSpec((BLK_, TW_), lambda i: (i, 0)),
            pl.BlockSpec((BLK_, 2 * H_), lambda i: (i, 0)),
        ],
        out_shape=[
            jax.ShapeDtypeStruct((NPAD_, TW_), jnp.float32),
            jax.ShapeDtypeStruct((NPAD_, 2 * H_), jnp.float32),
        ],
    )(x2d, W, As_bd, Ad_bd)


def _dense_stage_b(parts, rrep, b, g, be, W, As_bd, Ad_bd):
    return pl.pallas_call(
        _stage_b_body,
        grid=(NBLK_,),
        in_specs=[
            pl.BlockSpec((2, BLK_, TW_), lambda i: (0, i, 0)),
            pl.BlockSpec((H_, HC_), lambda i: (0, 0)),
            pl.BlockSpec((1, HC_), lambda i: (0, 0)),
            pl.BlockSpec((1, HC_), lambda i: (0, 0)),
            pl.BlockSpec((1, HC_), lambda i: (0, 0)),
            pl.BlockSpec((HC_, HC_), lambda i: (0, 0)),
            pl.BlockSpec((HC_, H_), lambda i: (0, 0)),
            pl.BlockSpec((HC_, H_), lambda i: (0, 0)),
        ],
        out_specs=[
            pl.BlockSpec((BLK_, TW_), lambda i: (i, 0)),
            pl.BlockSpec((BLK_, 2 * H_), lambda i: (i, 0)),
        ],
        out_shape=[
            jax.ShapeDtypeStruct((NPAD_, TW_), jnp.float32),
            jax.ShapeDtypeStruct((NPAD_, 2 * H_), jnp.float32),
        ],
    )(parts, rrep, b, g, be, W, As_bd, Ad_bd)


def _dense_stage_c(parts, batch3, rrep, b, g, be, l1w, l1b, gf, bf, l2w, l2b):
    return pl.pallas_call(
        _stage_c_body,
        grid=(NBLK_,),
        in_specs=[
            pl.BlockSpec((2, BLK_, TW_), lambda i: (0, i, 0)),
            pl.BlockSpec((1, 1, BLK_), lambda i: (i, 0, 0)),
            pl.BlockSpec((H_, HC_), lambda i: (0, 0)),
            pl.BlockSpec((1, HC_), lambda i: (0, 0)),
            pl.BlockSpec((1, HC_), lambda i: (0, 0)),
            pl.BlockSpec((1, HC_), lambda i: (0, 0)),
            pl.BlockSpec((HC_, C_), lambda i: (0, 0)),
            pl.BlockSpec((1, C_), lambda i: (0, 0)),
            pl.BlockSpec((1, C_), lambda i: (0, 0)),
            pl.BlockSpec((1, C_), lambda i: (0, 0)),
            pl.BlockSpec((C_, C_), lambda i: (0, 0)),
            pl.BlockSpec((1, C_), lambda i: (0, 0)),
        ],
        out_specs=pl.BlockSpec((G_, C_), lambda i: (0, 0)),
        out_shape=jax.ShapeDtypeStruct((G_, C_), jnp.float32),
        scratch_shapes=[
            pltpu.VMEM((G_, HC_), jnp.float32),
            pltpu.VMEM((G_, HC_), jnp.float32),
        ],
    )(parts, batch3, rrep, b, g, be, l1w, l1b, gf, bf, l2w, l2b)


def _edge_kernel_body(tsrc_hbm, tald_hbm, ei_hbm, z_hbm, out_hbm,
                      src0, src1, dst0, dst1,
                      rows0, rows1, ald0, ald1, msg0, acc,
                      sis0, sis1, sid0, sid1, sgr0, sgr1, sga0, sga1):
    c = lax.axis_index("c")
    s = lax.axis_index("s")
    wid = c * 16 + s
    base = wid * EW_

    SRC = (src0, src1)
    DST = (dst0, dst1)
    ROWS = (rows0, rows1)
    ALD = (ald0, ald1)
    SIS = (sis0, sis1)
    SID = (sid0, sid1)
    SGR = (sgr0, sgr1)
    SGA = (sga0, sga1)

    pltpu.sync_copy(z_hbm.at[pl.ds(s * ROWS_T_, ROWS_T_)],
                    acc.at[pl.ds(s * ROWS_T_, ROWS_T_)])
    plsc.subcore_barrier()

    def issue_ids(it, p):
        off = base + it * KCH_
        pltpu.async_copy(ei_hbm.at[pl.ds(off, KCH_)], SRC[p], SIS[p])
        pltpu.async_copy(ei_hbm.at[pl.ds(E_ + off, KCH_)], DST[p], SID[p])

    def wait_ids(p):
        pltpu.make_async_copy(ei_hbm.at[pl.ds(0, KCH_)], SRC[p], SIS[p]).wait()
        pltpu.make_async_copy(ei_hbm.at[pl.ds(0, KCH_)], DST[p], SID[p]).wait()

    def issue_gathers(p):
        pltpu.async_copy(tsrc_hbm.at[SRC[p]], ROWS[p], SGR[p])
        pltpu.async_copy(tald_hbm.at[DST[p]], ALD[p], SGA[p])

    def wait_gathers(p):
        pltpu.make_async_copy(tsrc_hbm.at[SRC[p]], ROWS[p], SGR[p]).wait()
        pltpu.make_async_copy(tald_hbm.at[DST[p]], ALD[p], SGA[p]).wait()

    def compute(p):
        rows_v, ald_v = ROWS[p], ALD[p]

        def edge(i, carry2):
            als = rows_v[i, pl.ds(HC_, 16)]
            aldv = ald_v[i, :]
            e = als + aldv
            e = jnp.maximum(e, 0.2 * e)
            ex = jnp.exp(e)
            msg0[i, pl.ds(HC_, 16)] = ex
            for h in range(H_):
                bc = ex[h]
                msg0[i, pl.ds(h * 16, 16)] = (
                    rows_v[i, pl.ds(h * 16, 16)] * bc)
            return carry2

        lax.fori_loop(0, KCH_, edge, 0, unroll=False)

    issue_ids(0, 0)
    issue_ids(1, 1)
    wait_ids(0)
    issue_gathers(0)

    def half(it, p, q):
        wait_gathers(p)

        @pl.when(it + 1 < NCH_)
        def _():
            wait_ids(q)
            issue_gathers(q)

        compute(p)
        pltpu.sync_copy(msg0, acc.at[DST[p]], add=True)

        @pl.when(it + 2 < NCH_)
        def _():
            issue_ids(it + 2, p)

    def round_(t, carry):
        half(2 * t, 0, 1)

        @pl.when(2 * t + 1 < NCH_)
        def _():
            half(2 * t + 1, 1, 0)

        return carry

    lax.fori_loop(0, (NCH_ + 1) // 2, round_, 0, unroll=False)
    plsc.subcore_barrier()
    pltpu.sync_copy(acc.at[pl.ds(s * ROWS_T_, ROWS_T_)],
                    out_hbm.at[c, pl.ds(s * ROWS_T_, ROWS_T_)])


def _edge_phase(tsrc, tald, edge_index, zeros_tab):
    mesh = plsc.VectorSubcoreMesh(core_axis_name="c", subcore_axis_name="s")
    fn = functools.partial(
        pl.kernel,
        mesh=mesh,
        compiler_params=pltpu.CompilerParams(use_tc_tiling_on_sc=False),
        out_type=jax.ShapeDtypeStruct((2, NPAD_, TW_), jnp.float32),
        scratch_types=(
            [pltpu.VMEM((KCH_,), jnp.int32)] * 4 +
            [pltpu.VMEM((KCH_, TW_), jnp.float32)] * 2 +
            [pltpu.VMEM((KCH_, 2 * H_), jnp.float32)] * 2 +
            [pltpu.VMEM((KCH_, TW_), jnp.float32)] +
            [pltpu.VMEM_SHARED((NPAD_, TW_), jnp.float32)] +
            [pltpu.SemaphoreType.DMA] * 8
        ),
    )(_edge_kernel_body)
    return fn(tsrc, tald, edge_index, zeros_tab)


def _block_diag_attn(a):
    return (a[:, :, None] * jnp.eye(H_, dtype=jnp.float32)[:, None, :]
            ).reshape(H_ * C_, H_)


def kernel(x, edge_index, edge_attr, batch, W0, a_src0, a_dst0, b0, g0, be0,
           W1, a_src1, a_dst1, b1, g1, be1, lin1_W, lin1_b, gf, bf,
           lin2_W, lin2_b):
    x2d = jnp.squeeze(x.astype(jnp.float32), axis=1)
    ei = edge_index.astype(jnp.int32).reshape(2 * E_)
    batch3 = batch.astype(jnp.int32).reshape(NBLK_, 1, BLK_)

    As0 = _block_diag_attn(a_src0)
    Ad0 = _block_diag_attn(a_dst0)
    As1 = _block_diag_attn(a_src1)
    Ad1 = _block_diag_attn(a_dst1)
    rrep = (jnp.arange(HC_)[None, :] // C_ ==
            jnp.arange(H_)[:, None]).astype(jnp.float32)
    zeros_tab = jnp.zeros((NPAD_, TW_), jnp.float32)

    r2 = lambda t: t.reshape(1, -1).astype(jnp.float32)

    tsrc0, tald0 = _dense_stage_a(x2d, W0, As0, Ad0)
    parts0 = _edge_phase(tsrc0, tald0, ei, zeros_tab)
    tsrc1, tald1 = _dense_stage_b(parts0, rrep, r2(b0), r2(g0), r2(be0),
                                  W1, As1, Ad1)
    parts1 = _edge_phase(tsrc1, tald1, ei, zeros_tab)
    return _dense_stage_c(parts1, batch3, rrep, r2(b1), r2(g1), r2(be1),
                          lin1_W, r2(lin1_b), r2(gf), r2(bf),
                          lin2_W, r2(lin2_b))


# speedup vs baseline: 123.3890x; 1.2463x over previous
import functools

import jax
import jax.numpy as jnp
from jax import lax
from jax.experimental import pallas as pl
from jax.experimental.pallas import tpu as pltpu
from jax.experimental.pallas import tpu_sc as plsc

N_ = 10000
E_ = 320000
F_ = 128
H_ = 8
C_ = 16
HC_ = 128
G_ = 64
TW_ = HC_ + 2 * H_

NBLK_ = 25
BLK_ = N_ // NBLK_

NWORK_ = 32
EW_ = E_ // NWORK_
KCH_ = 40
NCH_ = EW_ // KCH_
NPAD_ = 10240
ROWS_T_ = NPAD_ // 16


def _emit_tables(h, as_ref, ad_ref, tsrc_ref, tald_ref):
    als = jnp.dot(h, as_ref[...], preferred_element_type=jnp.float32)
    ald = jnp.dot(h, ad_ref[...], preferred_element_type=jnp.float32)
    z8 = jnp.zeros_like(als)
    tsrc_ref[...] = jnp.concatenate([h, als, z8], axis=1)
    tald_ref[...] = jnp.concatenate([ald, z8], axis=1)


def _stage_a_body(x_ref, w_ref, as_ref, ad_ref, tsrc_ref, tald_ref):
    h = jnp.dot(x_ref[...], w_ref[...], preferred_element_type=jnp.float32)
    _emit_tables(h, as_ref, ad_ref, tsrc_ref, tald_ref)


def _combine(p_ref, rrep_ref, b_ref, g_ref, be_ref):
    s = p_ref[0] + p_ref[1]
    num = s[:, :HC_]
    den8 = s[:, HC_:HC_ + H_]
    den = jnp.dot(den8, rrep_ref[...], preferred_element_type=jnp.float32)
    o = num / (den + 1e-16) + b_ref[...]
    m = jnp.mean(o, axis=1, keepdims=True)
    v = jnp.mean((o - m) ** 2, axis=1, keepdims=True)
    xn = (o - m) / jnp.sqrt(v + 1e-5) * g_ref[...] + be_ref[...]
    return jnp.maximum(xn, 0.0)


def _stage_b_body(p_ref, rrep_ref, b_ref, g_ref, be_ref, w_ref, as_ref,
                  ad_ref, tsrc_ref, tald_ref):
    xn = _combine(p_ref, rrep_ref, b_ref, g_ref, be_ref)
    h = jnp.dot(xn, w_ref[...], preferred_element_type=jnp.float32)
    _emit_tables(h, as_ref, ad_ref, tsrc_ref, tald_ref)


def _stage_c_body(p_ref, batch_ref, rrep_ref, b_ref, g_ref, be_ref,
                  l1w_ref, l1b_ref, gf_ref, bf_ref, l2w_ref, l2b_ref,
                  out_ref, pooled_scr, cnt_scr):
    i = pl.program_id(0)

    @pl.when(i == 0)
    def _():
        pooled_scr[...] = jnp.zeros_like(pooled_scr)
        cnt_scr[...] = jnp.zeros_like(cnt_scr)

    x2 = _combine(p_ref, rrep_ref, b_ref, g_ref, be_ref)
    bb = batch_ref[0]
    gid = lax.broadcasted_iota(jnp.int32, (G_, 1), 0)
    oh = (bb == gid).astype(jnp.float32)
    pooled_scr[...] += jnp.dot(oh, x2, preferred_element_type=jnp.float32)
    cnt_scr[...] += jnp.broadcast_to(
        jnp.sum(oh, axis=1, keepdims=True), (G_, HC_))

    @pl.when(i == NBLK_ - 1)
    def _():
        pm = pooled_scr[...] / jnp.maximum(cnt_scr[...], 1.0)
        a = jnp.dot(pm, l1w_ref[...], preferred_element_type=jnp.float32)
        a = a + l1b_ref[...]
        m = jnp.mean(a, axis=1, keepdims=True)
        v = jnp.mean((a - m) ** 2, axis=1, keepdims=True)
        a = (a - m) / jnp.sqrt(v + 1e-5) * gf_ref[...] + bf_ref[...]
        a = jnp.maximum(a, 0.0)
        out_ref[...] = (jnp.dot(a, l2w_ref[...],
                                preferred_element_type=jnp.float32)
                        + l2b_ref[...])


def _dense_stage_a(x2d, W, As_bd, Ad_bd):
    return pl.pallas_call(
        _stage_a_body,
        grid=(NBLK_,),
        in_specs=[
            pl.BlockSpec((BLK_, F_), lambda i: (i, 0)),
            pl.BlockSpec((F_, HC_), lambda i: (0, 0)),
            pl.BlockSpec((HC_, H_), lambda i: (0, 0)),
            pl.BlockSpec((HC_, H_), lambda i: (0, 0)),
        ],
        out_specs=[
            pl.BlockSpec((BLK_, TW_), lambda i: (i, 0)),
            pl.BlockSpec((BLK_, 2 * H_), lambda i: (i, 0)),
        ],
        out_shape=[
            jax.ShapeDtypeStruct((NPAD_, TW_), jnp.float32),
            jax.ShapeDtypeStruct((NPAD_, 2 * H_), jnp.float32),
        ],
    )(x2d, W, As_bd, Ad_bd)


def _dense_stage_b(parts, rrep, b, g, be, W, As_bd, Ad_bd):
    return pl.pallas_call(
        _stage_b_body,
        grid=(NBLK_,),
        in_specs=[
            pl.BlockSpec((2, BLK_, TW_), lambda i: (0, i, 0)),
            pl.BlockSpec((H_, HC_), lambda i: (0, 0)),
            pl.BlockSpec((1, HC_), lambda i: (0, 0)),
            pl.BlockSpec((1, HC_), lambda i: (0, 0)),
            pl.BlockSpec((1, HC_), lambda i: (0, 0)),
            pl.BlockSpec((HC_, HC_), lambda i: (0, 0)),
            pl.BlockSpec((HC_, H_), lambda i: (0, 0)),
            pl.BlockSpec((HC_, H_), lambda i: (0, 0)),
        ],
        out_specs=[
            pl.BlockSpec((BLK_, TW_), lambda i: (i, 0)),
            pl.BlockSpec((BLK_, 2 * H_), lambda i: (i, 0)),
        ],
        out_shape=[
            jax.ShapeDtypeStruct((NPAD_, TW_), jnp.float32),
            jax.ShapeDtypeStruct((NPAD_, 2 * H_), jnp.float32),
        ],
    )(parts, rrep, b, g, be, W, As_bd, Ad_bd)


def _dense_stage_c(parts, batch3, rrep, b, g, be, l1w, l1b, gf, bf, l2w, l2b):
    return pl.pallas_call(
        _stage_c_body,
        grid=(NBLK_,),
        in_specs=[
            pl.BlockSpec((2, BLK_, TW_), lambda i: (0, i, 0)),
            pl.BlockSpec((1, 1, BLK_), lambda i: (i, 0, 0)),
            pl.BlockSpec((H_, HC_), lambda i: (0, 0)),
            pl.BlockSpec((1, HC_), lambda i: (0, 0)),
            pl.BlockSpec((1, HC_), lambda i: (0, 0)),
            pl.BlockSpec((1, HC_), lambda i: (0, 0)),
            pl.BlockSpec((HC_, C_), lambda i: (0, 0)),
            pl.BlockSpec((1, C_), lambda i: (0, 0)),
            pl.BlockSpec((1, C_), lambda i: (0, 0)),
            pl.BlockSpec((1, C_), lambda i: (0, 0)),
            pl.BlockSpec((C_, C_), lambda i: (0, 0)),
            pl.BlockSpec((1, C_), lambda i: (0, 0)),
        ],
        out_specs=pl.BlockSpec((G_, C_), lambda i: (0, 0)),
        out_shape=jax.ShapeDtypeStruct((G_, C_), jnp.float32),
        scratch_shapes=[
            pltpu.VMEM((G_, HC_), jnp.float32),
            pltpu.VMEM((G_, HC_), jnp.float32),
        ],
    )(parts, batch3, rrep, b, g, be, l1w, l1b, gf, bf, l2w, l2b)


def _edge_kernel_body(tsrc_hbm, tald_hbm, ei_hbm, z_hbm, out_hbm,
                      src0, src1, dst0, dst1, sd0, sd1,
                      rows0, rows1, ald0, ald1, msg0, msg1, acc,
                      sis0, sis1, sid0, sid1, sgr0, sgr1, sga0, sga1,
                      ssc0, ssc1):
    c = lax.axis_index("c")
    s = lax.axis_index("s")
    wid = c * 16 + s
    base = wid * EW_

    SRC = (src0, src1)
    DST = (dst0, dst1)
    SD = (sd0, sd1)
    ROWS = (rows0, rows1)
    ALD = (ald0, ald1)
    MSG = (msg0, msg1)
    SIS = (sis0, sis1)
    SID = (sid0, sid1)
    SGR = (sgr0, sgr1)
    SGA = (sga0, sga1)
    SSC = (ssc0, ssc1)

    pltpu.sync_copy(z_hbm.at[pl.ds(s * ROWS_T_, ROWS_T_)],
                    acc.at[pl.ds(s * ROWS_T_, ROWS_T_)])
    pltpu.sync_copy(z_hbm.at[pl.ds(0, KCH_)], msg0)
    pltpu.sync_copy(z_hbm.at[pl.ds(0, KCH_)], msg1)
    plsc.subcore_barrier()

    def issue_ids(it, p):
        off = base + it * KCH_
        pltpu.async_copy(ei_hbm.at[pl.ds(off, KCH_)], SRC[p], SIS[p])
        pltpu.async_copy(ei_hbm.at[pl.ds(E_ + off, KCH_)], DST[p], SID[p])

    def wait_ids(p):
        pltpu.make_async_copy(ei_hbm.at[pl.ds(0, KCH_)], SRC[p], SIS[p]).wait()
        pltpu.make_async_copy(ei_hbm.at[pl.ds(0, KCH_)], DST[p], SID[p]).wait()

    def issue_gathers(p):
        pltpu.async_copy(tsrc_hbm.at[SRC[p]], ROWS[p], SGR[p])
        pltpu.async_copy(tald_hbm.at[DST[p]], ALD[p], SGA[p])

    def wait_gathers(p):
        pltpu.make_async_copy(tsrc_hbm.at[SRC[p]], ROWS[p], SGR[p]).wait()
        pltpu.make_async_copy(tald_hbm.at[DST[p]], ALD[p], SGA[p]).wait()

    def issue_scatter(p):
        pltpu.async_copy(MSG[p], acc.at[SD[p]], SSC[p], add=True)

    def wait_scatter(p):
        pltpu.make_async_copy(MSG[p], acc.at[SD[p]], SSC[p]).wait()

    def compute(p):
        rows_v, ald_v, msg_v = ROWS[p], ALD[p], MSG[p]

        def edge(i, carry2):
            als = rows_v[i, pl.ds(HC_, 16)]
            aldv = ald_v[i, :]
            e = als + aldv
            e = jnp.maximum(e, 0.2 * e)
            ex = jnp.exp(e)
            msg_v[i, pl.ds(HC_, 16)] = ex
            for h in range(H_):
                bc = ex[h]
                msg_v[i, pl.ds(h * 16, 16)] = (
                    rows_v[i, pl.ds(h * 16, 16)] * bc)
            return carry2

        lax.fori_loop(0, KCH_, edge, 0, unroll=False)

    _strips = list(range(0, KCH_ - 16, 16)) + [KCH_ - 16]

    def copy_dst(p):
        for j in _strips:
            SD[p][pl.ds(j, 16)] = DST[p][pl.ds(j, 16)]

    def zero_sd(p):
        for j in _strips:
            SD[p][pl.ds(j, 16)] = jnp.zeros((16,), jnp.int32)

    zero_sd(0)
    zero_sd(1)
    issue_scatter(0)
    issue_scatter(1)
    issue_ids(0, 0)
    issue_ids(1, 1)
    wait_ids(0)
    issue_gathers(0)

    def half(it, p, q):
        wait_gathers(p)
        wait_scatter(p)
        copy_dst(p)

        @pl.when(it + 1 < NCH_)
        def _():
            wait_ids(q)
            issue_gathers(q)

        @pl.when(it + 2 < NCH_)
        def _():
            issue_ids(it + 2, p)

        compute(p)
        issue_scatter(p)

    def round_(t, carry):
        half(2 * t, 0, 1)

        @pl.when(2 * t + 1 < NCH_)
        def _():
            half(2 * t + 1, 1, 0)

        return carry

    lax.fori_loop(0, (NCH_ + 1) // 2, round_, 0, unroll=False)
    wait_scatter(0)
    wait_scatter(1)
    plsc.subcore_barrier()
    pltpu.sync_copy(acc.at[pl.ds(s * ROWS_T_, ROWS_T_)],
                    out_hbm.at[c, pl.ds(s * ROWS_T_, ROWS_T_)])


def _edge_phase(tsrc, tald, edge_index, zeros_tab):
    mesh = plsc.VectorSubcoreMesh(core_axis_name="c", subcore_axis_name="s")
    fn = functools.partial(
        pl.kernel,
        mesh=mesh,
        compiler_params=pltpu.CompilerParams(use_tc_tiling_on_sc=False),
        out_type=jax.ShapeDtypeStruct((2, NPAD_, TW_), jnp.float32),
        scratch_types=(
            [pltpu.VMEM((KCH_,), jnp.int32)] * 6 +
            [pltpu.VMEM((KCH_, TW_), jnp.float32)] * 2 +
            [pltpu.VMEM((KCH_, 2 * H_), jnp.float32)] * 2 +
            [pltpu.VMEM((KCH_, TW_), jnp.float32)] * 2 +
            [pltpu.VMEM_SHARED((NPAD_, TW_), jnp.float32)] +
            [pltpu.SemaphoreType.DMA] * 10
        ),
    )(_edge_kernel_body)
    return fn(tsrc, tald, edge_index, zeros_tab)


def _block_diag_attn(a):
    return (a[:, :, None] * jnp.eye(H_, dtype=jnp.float32)[:, None, :]
            ).reshape(H_ * C_, H_)


def kernel(x, edge_index, edge_attr, batch, W0, a_src0, a_dst0, b0, g0, be0,
           W1, a_src1, a_dst1, b1, g1, be1, lin1_W, lin1_b, gf, bf,
           lin2_W, lin2_b):
    x2d = jnp.squeeze(x.astype(jnp.float32), axis=1)
    ei = edge_index.astype(jnp.int32).reshape(2 * E_)
    batch3 = batch.astype(jnp.int32).reshape(NBLK_, 1, BLK_)

    As0 = _block_diag_attn(a_src0)
    Ad0 = _block_diag_attn(a_dst0)
    As1 = _block_diag_attn(a_src1)
    Ad1 = _block_diag_attn(a_dst1)
    rrep = (jnp.arange(HC_)[None, :] // C_ ==
            jnp.arange(H_)[:, None]).astype(jnp.float32)
    zeros_tab = jnp.zeros((NPAD_, TW_), jnp.float32)

    r2 = lambda t: t.reshape(1, -1).astype(jnp.float32)

    tsrc0, tald0 = _dense_stage_a(x2d, W0, As0, Ad0)
    parts0 = _edge_phase(tsrc0, tald0, ei, zeros_tab)
    tsrc1, tald1 = _dense_stage_b(parts0, rrep, r2(b0), r2(g0), r2(be0),
                                  W1, As1, Ad1)
    parts1 = _edge_phase(tsrc1, tald1, ei, zeros_tab)
    return _dense_stage_c(parts1, batch3, rrep, r2(b1), r2(g1), r2(be1),
                          lin1_W, r2(lin1_b), r2(gf), r2(bf),
                          lin2_W, r2(lin2_b))


# speedup vs baseline: 123.4136x; 1.0002x over previous
import functools

import jax
import jax.numpy as jnp
from jax import lax
from jax.experimental import pallas as pl
from jax.experimental.pallas import tpu as pltpu
from jax.experimental.pallas import tpu_sc as plsc

N_ = 10000
E_ = 320000
F_ = 128
H_ = 8
C_ = 16
HC_ = 128
G_ = 64
TW_ = HC_ + 2 * H_

NBLK_ = 25
BLK_ = N_ // NBLK_

NWORK_ = 32
EW_ = E_ // NWORK_
KCH_ = 40
NCH_ = EW_ // KCH_
NPAD_ = 10240
ROWS_T_ = NPAD_ // 16


def _emit_tables(h, as_ref, ad_ref, tsrc_ref, tald_ref):
    als = jnp.dot(h, as_ref[...], preferred_element_type=jnp.float32)
    ald = jnp.dot(h, ad_ref[...], preferred_element_type=jnp.float32)
    z8 = jnp.zeros_like(als)
    tsrc_ref[...] = jnp.concatenate([h, als, z8], axis=1)
    tald_ref[...] = jnp.concatenate([ald, z8], axis=1)


def _stage_a_body(x_ref, w_ref, as_ref, ad_ref, tsrc_ref, tald_ref):
    h = jnp.dot(x_ref[...], w_ref[...], preferred_element_type=jnp.float32)
    _emit_tables(h, as_ref, ad_ref, tsrc_ref, tald_ref)


def _combine(p_ref, rrep_ref, b_ref, g_ref, be_ref):
    s = p_ref[0] + p_ref[1]
    num = s[:, :HC_]
    den8 = s[:, HC_:HC_ + H_]
    den = jnp.dot(den8, rrep_ref[...], preferred_element_type=jnp.float32)
    o = num / (den + 1e-16) + b_ref[...]
    m = jnp.mean(o, axis=1, keepdims=True)
    v = jnp.mean((o - m) ** 2, axis=1, keepdims=True)
    xn = (o - m) / jnp.sqrt(v + 1e-5) * g_ref[...] + be_ref[...]
    return jnp.maximum(xn, 0.0)


def _stage_b_body(p_ref, rrep_ref, b_ref, g_ref, be_ref, w_ref, as_ref,
                  ad_ref, tsrc_ref, tald_ref):
    xn = _combine(p_ref, rrep_ref, b_ref, g_ref, be_ref)
    h = jnp.dot(xn, w_ref[...], preferred_element_type=jnp.float32)
    _emit_tables(h, as_ref, ad_ref, tsrc_ref, tald_ref)


def _stage_c_body(p_ref, batch_ref, rrep_ref, b_ref, g_ref, be_ref,
                  l1w_ref, l1b_ref, gf_ref, bf_ref, l2w_ref, l2b_ref,
                  out_ref, pooled_scr, cnt_scr):
    i = pl.program_id(0)

    @pl.when(i == 0)
    def _():
        pooled_scr[...] = jnp.zeros_like(pooled_scr)
        cnt_scr[...] = jnp.zeros_like(cnt_scr)

    x2 = _combine(p_ref, rrep_ref, b_ref, g_ref, be_ref)
    bb = batch_ref[0]
    gid = lax.broadcasted_iota(jnp.int32, (G_, 1), 0)
    oh = (bb == gid).astype(jnp.float32)
    pooled_scr[...] += jnp.dot(oh, x2, preferred_element_type=jnp.float32)
    cnt_scr[...] += jnp.broadcast_to(
        jnp.sum(oh, axis=1, keepdims=True), (G_, HC_))

    @pl.when(i == NBLK_ - 1)
    def _():
        pm = pooled_scr[...] / jnp.maximum(cnt_scr[...], 1.0)
        a = jnp.dot(pm, l1w_ref[...], preferred_element_type=jnp.float32)
        a = a + l1b_ref[...]
        m = jnp.mean(a, axis=1, keepdims=True)
        v = jnp.mean((a - m) ** 2, axis=1, keepdims=True)
        a = (a - m) / jnp.sqrt(v + 1e-5) * gf_ref[...] + bf_ref[...]
        a = jnp.maximum(a, 0.0)
        out_ref[...] = (jnp.dot(a, l2w_ref[...],
                                preferred_element_type=jnp.float32)
                        + l2b_ref[...])


def _dense_stage_a(x2d, W, As_bd, Ad_bd):
    return pl.pallas_call(
        _stage_a_body,
        grid=(NBLK_,),
        in_specs=[
            pl.BlockSpec((BLK_, F_), lambda i: (i, 0)),
            pl.BlockSpec((F_, HC_), lambda i: (0, 0)),
            pl.BlockSpec((HC_, H_), lambda i: (0, 0)),
            pl.BlockSpec((HC_, H_), lambda i: (0, 0)),
        ],
        out_specs=[
            pl.BlockSpec((BLK_, TW_), lambda i: (i, 0)),
            pl.BlockSpec((BLK_, 2 * H_), lambda i: (i, 0)),
        ],
        out_shape=[
            jax.ShapeDtypeStruct((NPAD_, TW_), jnp.float32),
            jax.ShapeDtypeStruct((NPAD_, 2 * H_), jnp.float32),
        ],
    )(x2d, W, As_bd, Ad_bd)


def _dense_stage_b(parts, rrep, b, g, be, W, As_bd, Ad_bd):
    return pl.pallas_call(
        _stage_b_body,
        grid=(NBLK_,),
        in_specs=[
            pl.BlockSpec((2, BLK_, TW_), lambda i: (0, i, 0)),
            pl.BlockSpec((H_, HC_), lambda i: (0, 0)),
            pl.BlockSpec((1, HC_), lambda i: (0, 0)),
            pl.BlockSpec((1, HC_), lambda i: (0, 0)),
            pl.BlockSpec((1, HC_), lambda i: (0, 0)),
            pl.BlockSpec((HC_, HC_), lambda i: (0, 0)),
            pl.BlockSpec((HC_, H_), lambda i: (0, 0)),
            pl.BlockSpec((HC_, H_), lambda i: (0, 0)),
        ],
        out_specs=[
            pl.BlockSpec((BLK_, TW_), lambda i: (i, 0)),
            pl.BlockSpec((BLK_, 2 * H_), lambda i: (i, 0)),
        ],
        out_shape=[
            jax.ShapeDtypeStruct((NPAD_, TW_), jnp.float32),
            jax.ShapeDtypeStruct((NPAD_, 2 * H_), jnp.float32),
        ],
    )(parts, rrep, b, g, be, W, As_bd, Ad_bd)


def _dense_stage_c(parts, batch3, rrep, b, g, be, l1w, l1b, gf, bf, l2w, l2b):
    return pl.pallas_call(
        _stage_c_body,
        grid=(NBLK_,),
        in_specs=[
            pl.BlockSpec((2, BLK_, TW_), lambda i: (0, i, 0)),
            pl.BlockSpec((1, 1, BLK_), lambda i: (i, 0, 0)),
            pl.BlockSpec((H_, HC_), lambda i: (0, 0)),
            pl.BlockSpec((1, HC_), lambda i: (0, 0)),
            pl.BlockSpec((1, HC_), lambda i: (0, 0)),
            pl.BlockSpec((1, HC_), lambda i: (0, 0)),
            pl.BlockSpec((HC_, C_), lambda i: (0, 0)),
            pl.BlockSpec((1, C_), lambda i: (0, 0)),
            pl.BlockSpec((1, C_), lambda i: (0, 0)),
            pl.BlockSpec((1, C_), lambda i: (0, 0)),
            pl.BlockSpec((C_, C_), lambda i: (0, 0)),
            pl.BlockSpec((1, C_), lambda i: (0, 0)),
        ],
        out_specs=pl.BlockSpec((G_, C_), lambda i: (0, 0)),
        out_shape=jax.ShapeDtypeStruct((G_, C_), jnp.float32),
        scratch_shapes=[
            pltpu.VMEM((G_, HC_), jnp.float32),
            pltpu.VMEM((G_, HC_), jnp.float32),
        ],
    )(parts, batch3, rrep, b, g, be, l1w, l1b, gf, bf, l2w, l2b)


def _edge_kernel_body(tsrc_hbm, tald_hbm, ei_hbm, z_hbm, out_hbm,
                      src0, src1, dst0, dst1, sd0, sd1,
                      rows0, rows1, ald0, ald1, msg0, msg1, acc,
                      sis0, sis1, sid0, sid1, sgr0, sgr1, sga0, sga1,
                      ssc0, ssc1):
    c = lax.axis_index("c")
    s = lax.axis_index("s")
    wid = c * 16 + s
    base = wid * EW_

    SRC = (src0, src1)
    DST = (dst0, dst1)
    SD = (sd0, sd1)
    ROWS = (rows0, rows1)
    ALD = (ald0, ald1)
    MSG = (msg0, msg1)
    SIS = (sis0, sis1)
    SID = (sid0, sid1)
    SGR = (sgr0, sgr1)
    SGA = (sga0, sga1)
    SSC = (ssc0, ssc1)

    pltpu.sync_copy(z_hbm.at[pl.ds(s * ROWS_T_, ROWS_T_)],
                    acc.at[pl.ds(s * ROWS_T_, ROWS_T_)])
    pltpu.sync_copy(z_hbm.at[pl.ds(0, KCH_)], msg0)
    pltpu.sync_copy(z_hbm.at[pl.ds(0, KCH_)], msg1)
    plsc.subcore_barrier()

    def issue_ids(it, p):
        off = base + it * KCH_
        pltpu.async_copy(ei_hbm.at[pl.ds(off, KCH_)], SRC[p], SIS[p])
        pltpu.async_copy(ei_hbm.at[pl.ds(E_ + off, KCH_)], DST[p], SID[p])

    def wait_ids(p):
        pltpu.make_async_copy(ei_hbm.at[pl.ds(0, KCH_)], SRC[p], SIS[p]).wait()
        pltpu.make_async_copy(ei_hbm.at[pl.ds(0, KCH_)], DST[p], SID[p]).wait()

    def issue_gathers(p):
        pltpu.async_copy(tsrc_hbm.at[SRC[p]], ROWS[p], SGR[p])
        pltpu.async_copy(tald_hbm.at[DST[p]], ALD[p], SGA[p])

    def wait_gathers(p):
        pltpu.make_async_copy(tsrc_hbm.at[SRC[p]], ROWS[p], SGR[p]).wait()
        pltpu.make_async_copy(tald_hbm.at[DST[p]], ALD[p], SGA[p]).wait()

    def issue_scatter(p):
        pltpu.async_copy(MSG[p], acc.at[SD[p]], SSC[p], add=True)

    def wait_scatter(p):
        pltpu.make_async_copy(MSG[p], acc.at[SD[p]], SSC[p]).wait()

    def compute(p):
        rows_v, ald_v, msg_v = ROWS[p], ALD[p], MSG[p]

        @plsc.parallel_loop(0, KCH_, unroll=2)
        def edge(i):
            als = rows_v[i, pl.ds(HC_, 16)]
            aldv = ald_v[i, :]
            e = als + aldv
            e = jnp.maximum(e, 0.2 * e)
            ex = jnp.exp(e)
            msg_v[i, pl.ds(HC_, 16)] = ex
            for h in range(H_):
                bc = ex[h]
                msg_v[i, pl.ds(h * 16, 16)] = (
                    rows_v[i, pl.ds(h * 16, 16)] * bc)

    _strips = list(range(0, KCH_ - 16, 16)) + [KCH_ - 16]

    def copy_dst(p):
        for j in _strips:
            SD[p][pl.ds(j, 16)] = DST[p][pl.ds(j, 16)]

    def zero_sd(p):
        for j in _strips:
            SD[p][pl.ds(j, 16)] = jnp.zeros((16,), jnp.int32)

    zero_sd(0)
    zero_sd(1)
    issue_scatter(0)
    issue_scatter(1)
    issue_ids(0, 0)
    issue_ids(1, 1)
    wait_ids(0)
    issue_gathers(0)

    def half(it, p, q):
        wait_gathers(p)
        wait_scatter(p)
        copy_dst(p)

        @pl.when(it + 1 < NCH_)
        def _():
            wait_ids(q)
            issue_gathers(q)

        @pl.when(it + 2 < NCH_)
        def _():
            issue_ids(it + 2, p)

        compute(p)
        issue_scatter(p)

    def round_(t, carry):
        half(2 * t, 0, 1)

        @pl.when(2 * t + 1 < NCH_)
        def _():
            half(2 * t + 1, 1, 0)

        return carry

    lax.fori_loop(0, (NCH_ + 1) // 2, round_, 0, unroll=False)
    wait_scatter(0)
    wait_scatter(1)
    plsc.subcore_barrier()
    pltpu.sync_copy(acc.at[pl.ds(s * ROWS_T_, ROWS_T_)],
                    out_hbm.at[c, pl.ds(s * ROWS_T_, ROWS_T_)])


def _edge_phase(tsrc, tald, edge_index, zeros_tab):
    mesh = plsc.VectorSubcoreMesh(core_axis_name="c", subcore_axis_name="s")
    fn = functools.partial(
        pl.kernel,
        mesh=mesh,
        compiler_params=pltpu.CompilerParams(use_tc_tiling_on_sc=False),
        out_type=jax.ShapeDtypeStruct((2, NPAD_, TW_), jnp.float32),
        scratch_types=(
            [pltpu.VMEM((KCH_,), jnp.int32)] * 6 +
            [pltpu.VMEM((KCH_, TW_), jnp.float32)] * 2 +
            [pltpu.VMEM((KCH_, 2 * H_), jnp.float32)] * 2 +
            [pltpu.VMEM((KCH_, TW_), jnp.float32)] * 2 +
            [pltpu.VMEM_SHARED((NPAD_, TW_), jnp.float32)] +
            [pltpu.SemaphoreType.DMA] * 10
        ),
    )(_edge_kernel_body)
    return fn(tsrc, tald, edge_index, zeros_tab)


def _block_diag_attn(a):
    return (a[:, :, None] * jnp.eye(H_, dtype=jnp.float32)[:, None, :]
            ).reshape(H_ * C_, H_)


def kernel(x, edge_index, edge_attr, batch, W0, a_src0, a_dst0, b0, g0, be0,
           W1, a_src1, a_dst1, b1, g1, be1, lin1_W, lin1_b, gf, bf,
           lin2_W, lin2_b):
    x2d = jnp.squeeze(x.astype(jnp.float32), axis=1)
    ei = edge_index.astype(jnp.int32).reshape(2 * E_)
    batch3 = batch.astype(jnp.int32).reshape(NBLK_, 1, BLK_)

    As0 = _block_diag_attn(a_src0)
    Ad0 = _block_diag_attn(a_dst0)
    As1 = _block_diag_attn(a_src1)
    Ad1 = _block_diag_attn(a_dst1)
    rrep = (jnp.arange(HC_)[None, :] // C_ ==
            jnp.arange(H_)[:, None]).astype(jnp.float32)
    zeros_tab = jnp.zeros((NPAD_, TW_), jnp.float32)

    r2 = lambda t: t.reshape(1, -1).astype(jnp.float32)

    tsrc0, tald0 = _dense_stage_a(x2d, W0, As0, Ad0)
    parts0 = _edge_phase(tsrc0, tald0, ei, zeros_tab)
    tsrc1, tald1 = _dense_stage_b(parts0, rrep, r2(b0), r2(g0), r2(be0),
                                  W1, As1, Ad1)
    parts1 = _edge_phase(tsrc1, tald1, ei, zeros_tab)
    return _dense_stage_c(parts1, batch3, rrep, r2(b1), r2(g1), r2(be1),
                          lin1_W, r2(lin1_b), r2(gf), r2(bf),
                          lin2_W, r2(lin2_b))


# speedup vs baseline: 168.4702x; 1.3651x over previous
import functools

import jax
import jax.numpy as jnp
from jax import lax
from jax.experimental import pallas as pl
from jax.experimental.pallas import tpu as pltpu
from jax.experimental.pallas import tpu_sc as plsc

N_ = 10000
E_ = 320000
F_ = 128
H_ = 8
C_ = 16
HC_ = 128
G_ = 64
TW_ = HC_ + 2 * H_

NBLK_ = 25
BLK_ = N_ // NBLK_

NWORK_ = 32
EW_ = E_ // NWORK_
KCH_ = 40
NCH_ = EW_ // KCH_
NPAD_ = 10240
ROWS_T_ = NPAD_ // 16


def _emit_tables(h, as_ref, ad_ref, tsrc_ref, tald_ref):
    als = jnp.dot(h, as_ref[...], preferred_element_type=jnp.float32)
    ald = jnp.dot(h, ad_ref[...], preferred_element_type=jnp.float32)
    z8 = jnp.zeros_like(als)
    tsrc_ref[...] = jnp.concatenate([h, als, z8], axis=1)
    tald_ref[...] = jnp.concatenate([ald, z8], axis=1)


def _stage_a_body(x_ref, w_ref, as_ref, ad_ref, tsrc_ref, tald_ref):
    h = jnp.dot(x_ref[...], w_ref[...], preferred_element_type=jnp.float32)
    _emit_tables(h, as_ref, ad_ref, tsrc_ref, tald_ref)


def _combine(p_ref, rrep_ref, b_ref, g_ref, be_ref):
    s = p_ref[0] + p_ref[1]
    num = s[:, :HC_]
    den8 = s[:, HC_:HC_ + H_]
    den = jnp.dot(den8, rrep_ref[...], preferred_element_type=jnp.float32)
    o = num / (den + 1e-16) + b_ref[...]
    m = jnp.mean(o, axis=1, keepdims=True)
    v = jnp.mean((o - m) ** 2, axis=1, keepdims=True)
    xn = (o - m) / jnp.sqrt(v + 1e-5) * g_ref[...] + be_ref[...]
    return jnp.maximum(xn, 0.0)


def _stage_b_body(p_ref, rrep_ref, b_ref, g_ref, be_ref, w_ref, as_ref,
                  ad_ref, tsrc_ref, tald_ref):
    xn = _combine(p_ref, rrep_ref, b_ref, g_ref, be_ref)
    h = jnp.dot(xn, w_ref[...], preferred_element_type=jnp.float32)
    _emit_tables(h, as_ref, ad_ref, tsrc_ref, tald_ref)


def _stage_c_body(p_ref, batch_ref, rrep_ref, b_ref, g_ref, be_ref,
                  l1w_ref, l1b_ref, gf_ref, bf_ref, l2w_ref, l2b_ref,
                  out_ref, pooled_scr, cnt_scr):
    i = pl.program_id(0)

    @pl.when(i == 0)
    def _():
        pooled_scr[...] = jnp.zeros_like(pooled_scr)
        cnt_scr[...] = jnp.zeros_like(cnt_scr)

    x2 = _combine(p_ref, rrep_ref, b_ref, g_ref, be_ref)
    bb = batch_ref[0]
    gid = lax.broadcasted_iota(jnp.int32, (G_, 1), 0)
    oh = (bb == gid).astype(jnp.float32)
    pooled_scr[...] += jnp.dot(oh, x2, preferred_element_type=jnp.float32)
    cnt_scr[...] += jnp.broadcast_to(
        jnp.sum(oh, axis=1, keepdims=True), (G_, HC_))

    @pl.when(i == NBLK_ - 1)
    def _():
        pm = pooled_scr[...] / jnp.maximum(cnt_scr[...], 1.0)
        a = jnp.dot(pm, l1w_ref[...], preferred_element_type=jnp.float32)
        a = a + l1b_ref[...]
        m = jnp.mean(a, axis=1, keepdims=True)
        v = jnp.mean((a - m) ** 2, axis=1, keepdims=True)
        a = (a - m) / jnp.sqrt(v + 1e-5) * gf_ref[...] + bf_ref[...]
        a = jnp.maximum(a, 0.0)
        out_ref[...] = (jnp.dot(a, l2w_ref[...],
                                preferred_element_type=jnp.float32)
                        + l2b_ref[...])


def _dense_stage_a(x2d, W, As_bd, Ad_bd):
    return pl.pallas_call(
        _stage_a_body,
        grid=(NBLK_,),
        in_specs=[
            pl.BlockSpec((BLK_, F_), lambda i: (i, 0)),
            pl.BlockSpec((F_, HC_), lambda i: (0, 0)),
            pl.BlockSpec((HC_, H_), lambda i: (0, 0)),
            pl.BlockSpec((HC_, H_), lambda i: (0, 0)),
        ],
        out_specs=[
            pl.BlockSpec((BLK_, TW_), lambda i: (i, 0)),
            pl.BlockSpec((BLK_, 2 * H_), lambda i: (i, 0)),
        ],
        out_shape=[
            jax.ShapeDtypeStruct((NPAD_, TW_), jnp.float32),
            jax.ShapeDtypeStruct((NPAD_, 2 * H_), jnp.float32),
        ],
    )(x2d, W, As_bd, Ad_bd)


def _dense_stage_b(parts, rrep, b, g, be, W, As_bd, Ad_bd):
    return pl.pallas_call(
        _stage_b_body,
        grid=(NBLK_,),
        in_specs=[
            pl.BlockSpec((2, BLK_, TW_), lambda i: (0, i, 0)),
            pl.BlockSpec((H_, HC_), lambda i: (0, 0)),
            pl.BlockSpec((1, HC_), lambda i: (0, 0)),
            pl.BlockSpec((1, HC_), lambda i: (0, 0)),
            pl.BlockSpec((1, HC_), lambda i: (0, 0)),
            pl.BlockSpec((HC_, HC_), lambda i: (0, 0)),
            pl.BlockSpec((HC_, H_), lambda i: (0, 0)),
            pl.BlockSpec((HC_, H_), lambda i: (0, 0)),
        ],
        out_specs=[
            pl.BlockSpec((BLK_, TW_), lambda i: (i, 0)),
            pl.BlockSpec((BLK_, 2 * H_), lambda i: (i, 0)),
        ],
        out_shape=[
            jax.ShapeDtypeStruct((NPAD_, TW_), jnp.float32),
            jax.ShapeDtypeStruct((NPAD_, 2 * H_), jnp.float32),
        ],
    )(parts, rrep, b, g, be, W, As_bd, Ad_bd)


def _dense_stage_c(parts, batch3, rrep, b, g, be, l1w, l1b, gf, bf, l2w, l2b):
    return pl.pallas_call(
        _stage_c_body,
        grid=(NBLK_,),
        in_specs=[
            pl.BlockSpec((2, BLK_, TW_), lambda i: (0, i, 0)),
            pl.BlockSpec((1, 1, BLK_), lambda i: (i, 0, 0)),
            pl.BlockSpec((H_, HC_), lambda i: (0, 0)),
            pl.BlockSpec((1, HC_), lambda i: (0, 0)),
            pl.BlockSpec((1, HC_), lambda i: (0, 0)),
            pl.BlockSpec((1, HC_), lambda i: (0, 0)),
            pl.BlockSpec((HC_, C_), lambda i: (0, 0)),
            pl.BlockSpec((1, C_), lambda i: (0, 0)),
            pl.BlockSpec((1, C_), lambda i: (0, 0)),
            pl.BlockSpec((1, C_), lambda i: (0, 0)),
            pl.BlockSpec((C_, C_), lambda i: (0, 0)),
            pl.BlockSpec((1, C_), lambda i: (0, 0)),
        ],
        out_specs=pl.BlockSpec((G_, C_), lambda i: (0, 0)),
        out_shape=jax.ShapeDtypeStruct((G_, C_), jnp.float32),
        scratch_shapes=[
            pltpu.VMEM((G_, HC_), jnp.float32),
            pltpu.VMEM((G_, HC_), jnp.float32),
        ],
    )(parts, batch3, rrep, b, g, be, l1w, l1b, gf, bf, l2w, l2b)


def _edge_kernel_body(tsrc_hbm, tald_hbm, ei_hbm, z_hbm, out_hbm,
                      src0, src1, src2, src3, dst0, dst1, dst2, dst3,
                      sd0, sd1,
                      rows0, rows1, rows2, rows3, ald0, ald1, ald2, ald3,
                      msg0, msg1, acc,
                      sis0, sis1, sis2, sis3, sid0, sid1, sid2, sid3,
                      sgr0, sgr1, sgr2, sgr3, sga0, sga1, sga2, sga3,
                      ssc0, ssc1):
    c = lax.axis_index("c")
    s = lax.axis_index("s")
    wid = c * 16 + s
    base = wid * EW_

    SRC = (src0, src1, src2, src3)
    DST = (dst0, dst1, dst2, dst3)
    SD = (sd0, sd1)
    ROWS = (rows0, rows1, rows2, rows3)
    ALD = (ald0, ald1, ald2, ald3)
    MSG = (msg0, msg1)
    SIS = (sis0, sis1, sis2, sis3)
    SID = (sid0, sid1, sid2, sid3)
    SGR = (sgr0, sgr1, sgr2, sgr3)
    SGA = (sga0, sga1, sga2, sga3)
    SSC = (ssc0, ssc1)

    pltpu.sync_copy(z_hbm.at[pl.ds(s * ROWS_T_, ROWS_T_)],
                    acc.at[pl.ds(s * ROWS_T_, ROWS_T_)])
    pltpu.sync_copy(z_hbm.at[pl.ds(0, KCH_)], msg0)
    pltpu.sync_copy(z_hbm.at[pl.ds(0, KCH_)], msg1)
    plsc.subcore_barrier()

    def issue_ids(it, b):
        off = base + it * KCH_
        pltpu.async_copy(ei_hbm.at[pl.ds(off, KCH_)], SRC[b], SIS[b])
        pltpu.async_copy(ei_hbm.at[pl.ds(E_ + off, KCH_)], DST[b], SID[b])

    def wait_ids(b):
        pltpu.make_async_copy(ei_hbm.at[pl.ds(0, KCH_)], SRC[b], SIS[b]).wait()
        pltpu.make_async_copy(ei_hbm.at[pl.ds(0, KCH_)], DST[b], SID[b]).wait()

    def issue_gathers(b):
        pltpu.async_copy(tsrc_hbm.at[SRC[b]], ROWS[b], SGR[b])
        pltpu.async_copy(tald_hbm.at[DST[b]], ALD[b], SGA[b])

    def wait_gathers(b):
        pltpu.make_async_copy(tsrc_hbm.at[SRC[b]], ROWS[b], SGR[b]).wait()
        pltpu.make_async_copy(tald_hbm.at[DST[b]], ALD[b], SGA[b]).wait()

    def issue_scatter(m):
        pltpu.async_copy(MSG[m], acc.at[SD[m]], SSC[m], add=True)

    def wait_scatter(m):
        pltpu.make_async_copy(MSG[m], acc.at[SD[m]], SSC[m]).wait()

    def compute(b, m):
        rows_v, ald_v, msg_v = ROWS[b], ALD[b], MSG[m]

        @plsc.parallel_loop(0, KCH_, unroll=2)
        def edge(i):
            als = rows_v[i, pl.ds(HC_, 16)]
            aldv = ald_v[i, :]
            e = als + aldv
            e = jnp.maximum(e, 0.2 * e)
            ex = jnp.exp(e)
            msg_v[i, pl.ds(HC_, 16)] = ex
            for h in range(H_):
                bc = ex[h]
                msg_v[i, pl.ds(h * 16, 16)] = (
                    rows_v[i, pl.ds(h * 16, 16)] * bc)

    _strips = list(range(0, KCH_ - 16, 16)) + [KCH_ - 16]

    def copy_dst(b, m):
        for j in _strips:
            SD[m][pl.ds(j, 16)] = DST[b][pl.ds(j, 16)]

    def zero_sd(m):
        for j in _strips:
            SD[m][pl.ds(j, 16)] = jnp.zeros((16,), jnp.int32)

    zero_sd(0)
    zero_sd(1)
    issue_scatter(0)
    issue_scatter(1)
    for b in range(4):
        issue_ids(b, b)
    wait_ids(0)
    issue_gathers(0)
    wait_ids(1)
    issue_gathers(1)

    def quarter(it, b, m):
        wait_gathers(b)
        wait_scatter(m)
        copy_dst(b, m)

        @pl.when(it + 2 < NCH_)
        def _():
            wait_ids((b + 2) % 4)
            issue_gathers((b + 2) % 4)

        @pl.when(it + 4 < NCH_)
        def _():
            issue_ids(it + 4, b)

        compute(b, m)
        issue_scatter(m)

    def round_(t, carry):
        for q in range(4):
            it = 4 * t + q

            @pl.when(it < NCH_)
            def _():
                quarter(it, q, q % 2)

        return carry

    lax.fori_loop(0, (NCH_ + 3) // 4, round_, 0, unroll=False)
    wait_scatter(0)
    wait_scatter(1)
    plsc.subcore_barrier()
    pltpu.sync_copy(acc.at[pl.ds(s * ROWS_T_, ROWS_T_)],
                    out_hbm.at[c, pl.ds(s * ROWS_T_, ROWS_T_)])


def _edge_phase(tsrc, tald, edge_index, zeros_tab):
    mesh = plsc.VectorSubcoreMesh(core_axis_name="c", subcore_axis_name="s")
    fn = functools.partial(
        pl.kernel,
        mesh=mesh,
        compiler_params=pltpu.CompilerParams(use_tc_tiling_on_sc=False),
        out_type=jax.ShapeDtypeStruct((2, NPAD_, TW_), jnp.float32),
        scratch_types=(
            [pltpu.VMEM((KCH_,), jnp.int32)] * 10 +
            [pltpu.VMEM((KCH_, TW_), jnp.float32)] * 4 +
            [pltpu.VMEM((KCH_, 2 * H_), jnp.float32)] * 4 +
            [pltpu.VMEM((KCH_, TW_), jnp.float32)] * 2 +
            [pltpu.VMEM_SHARED((NPAD_, TW_), jnp.float32)] +
            [pltpu.SemaphoreType.DMA] * 18
        ),
    )(_edge_kernel_body)
    return fn(tsrc, tald, edge_index, zeros_tab)


def _block_diag_attn(a):
    return (a[:, :, None] * jnp.eye(H_, dtype=jnp.float32)[:, None, :]
            ).reshape(H_ * C_, H_)


def kernel(x, edge_index, edge_attr, batch, W0, a_src0, a_dst0, b0, g0, be0,
           W1, a_src1, a_dst1, b1, g1, be1, lin1_W, lin1_b, gf, bf,
           lin2_W, lin2_b):
    x2d = jnp.squeeze(x.astype(jnp.float32), axis=1)
    ei = edge_index.astype(jnp.int32).reshape(2 * E_)
    batch3 = batch.astype(jnp.int32).reshape(NBLK_, 1, BLK_)

    As0 = _block_diag_attn(a_src0)
    Ad0 = _block_diag_attn(a_dst0)
    As1 = _block_diag_attn(a_src1)
    Ad1 = _block_diag_attn(a_dst1)
    rrep = (jnp.arange(HC_)[None, :] // C_ ==
            jnp.arange(H_)[:, None]).astype(jnp.float32)
    zeros_tab = jnp.zeros((NPAD_, TW_), jnp.float32)

    r2 = lambda t: t.reshape(1, -1).astype(jnp.float32)

    tsrc0, tald0 = _dense_stage_a(x2d, W0, As0, Ad0)
    parts0 = _edge_phase(tsrc0, tald0, ei, zeros_tab)
    tsrc1, tald1 = _dense_stage_b(parts0, rrep, r2(b0), r2(g0), r2(be0),
                                  W1, As1, Ad1)
    parts1 = _edge_phase(tsrc1, tald1, ei, zeros_tab)
    return _dense_stage_c(parts1, batch3, rrep, r2(b1), r2(g1), r2(be1),
                          lin1_W, r2(lin1_b), r2(gf), r2(bf),
                          lin2_W, r2(lin2_b))


# speedup vs baseline: 186.6052x; 1.1076x over previous
import functools

import jax
import jax.numpy as jnp
from jax import lax
from jax.experimental import pallas as pl
from jax.experimental.pallas import tpu as pltpu
from jax.experimental.pallas import tpu_sc as plsc

N_ = 10000
E_ = 320000
F_ = 128
H_ = 8
C_ = 16
HC_ = 128
G_ = 64
TW_ = HC_ + 2 * H_

NBLK_ = 25
BLK_ = N_ // NBLK_

NWORK_ = 32
EW_ = E_ // NWORK_
KCH_ = 40
NCH_ = EW_ // KCH_
NPAD_ = 10240
ROWS_T_ = NPAD_ // 16


def _emit_tables(h, as_ref, ad_ref, h_ref, als_ref, ald_ref):
    als = jnp.dot(h, as_ref[...], preferred_element_type=jnp.float32)
    ald = jnp.dot(h, ad_ref[...], preferred_element_type=jnp.float32)
    z8 = jnp.zeros_like(als)
    h_ref[...] = h
    als_ref[...] = jnp.concatenate([als, z8], axis=1)
    ald_ref[...] = jnp.concatenate([ald, z8], axis=1)


def _stage_a_body(x_ref, w_ref, as_ref, ad_ref, h_ref, als_ref, ald_ref):
    h = jnp.dot(x_ref[...], w_ref[...], preferred_element_type=jnp.float32)
    _emit_tables(h, as_ref, ad_ref, h_ref, als_ref, ald_ref)


def _combine(pn_ref, pd_ref, rrep_ref, b_ref, g_ref, be_ref):
    num = pn_ref[0] + pn_ref[1]
    den8 = (pd_ref[0] + pd_ref[1])[:, :H_]
    den = jnp.dot(den8, rrep_ref[...], preferred_element_type=jnp.float32)
    o = num / (den + 1e-16) + b_ref[...]
    m = jnp.mean(o, axis=1, keepdims=True)
    v = jnp.mean((o - m) ** 2, axis=1, keepdims=True)
    xn = (o - m) / jnp.sqrt(v + 1e-5) * g_ref[...] + be_ref[...]
    return jnp.maximum(xn, 0.0)


def _stage_b_body(pn_ref, pd_ref, rrep_ref, b_ref, g_ref, be_ref, w_ref,
                  as_ref, ad_ref, h_ref, als_ref, ald_ref):
    xn = _combine(pn_ref, pd_ref, rrep_ref, b_ref, g_ref, be_ref)
    h = jnp.dot(xn, w_ref[...], preferred_element_type=jnp.float32)
    _emit_tables(h, as_ref, ad_ref, h_ref, als_ref, ald_ref)


def _stage_c_body(pn_ref, pd_ref, batch_ref, rrep_ref, b_ref, g_ref, be_ref,
                  l1w_ref, l1b_ref, gf_ref, bf_ref, l2w_ref, l2b_ref,
                  out_ref, pooled_scr, cnt_scr):
    i = pl.program_id(0)

    @pl.when(i == 0)
    def _():
        pooled_scr[...] = jnp.zeros_like(pooled_scr)
        cnt_scr[...] = jnp.zeros_like(cnt_scr)

    x2 = _combine(pn_ref, pd_ref, rrep_ref, b_ref, g_ref, be_ref)
    bb = batch_ref[0]
    gid = lax.broadcasted_iota(jnp.int32, (G_, 1), 0)
    oh = (bb == gid).astype(jnp.float32)
    pooled_scr[...] += jnp.dot(oh, x2, preferred_element_type=jnp.float32)
    cnt_scr[...] += jnp.broadcast_to(
        jnp.sum(oh, axis=1, keepdims=True), (G_, HC_))

    @pl.when(i == NBLK_ - 1)
    def _():
        pm = pooled_scr[...] / jnp.maximum(cnt_scr[...], 1.0)
        a = jnp.dot(pm, l1w_ref[...], preferred_element_type=jnp.float32)
        a = a + l1b_ref[...]
        m = jnp.mean(a, axis=1, keepdims=True)
        v = jnp.mean((a - m) ** 2, axis=1, keepdims=True)
        a = (a - m) / jnp.sqrt(v + 1e-5) * gf_ref[...] + bf_ref[...]
        a = jnp.maximum(a, 0.0)
        out_ref[...] = (jnp.dot(a, l2w_ref[...],
                                preferred_element_type=jnp.float32)
                        + l2b_ref[...])


_TABLE_SPECS = [
    pl.BlockSpec((BLK_, HC_), lambda i: (i, 0)),
    pl.BlockSpec((BLK_, 2 * H_), lambda i: (i, 0)),
    pl.BlockSpec((BLK_, 2 * H_), lambda i: (i, 0)),
]
_TABLE_SHAPES = [
    jax.ShapeDtypeStruct((NPAD_, HC_), jnp.float32),
    jax.ShapeDtypeStruct((NPAD_, 2 * H_), jnp.float32),
    jax.ShapeDtypeStruct((NPAD_, 2 * H_), jnp.float32),
]


def _dense_stage_a(x2d, W, As_bd, Ad_bd):
    return pl.pallas_call(
        _stage_a_body,
        grid=(NBLK_,),
        in_specs=[
            pl.BlockSpec((BLK_, F_), lambda i: (i, 0)),
            pl.BlockSpec((F_, HC_), lambda i: (0, 0)),
            pl.BlockSpec((HC_, H_), lambda i: (0, 0)),
            pl.BlockSpec((HC_, H_), lambda i: (0, 0)),
        ],
        out_specs=_TABLE_SPECS,
        out_shape=_TABLE_SHAPES,
    )(x2d, W, As_bd, Ad_bd)


def _dense_stage_b(pnum, pden, rrep, b, g, be, W, As_bd, Ad_bd):
    return pl.pallas_call(
        _stage_b_body,
        grid=(NBLK_,),
        in_specs=[
            pl.BlockSpec((2, BLK_, HC_), lambda i: (0, i, 0)),
            pl.BlockSpec((2, BLK_, 2 * H_), lambda i: (0, i, 0)),
            pl.BlockSpec((H_, HC_), lambda i: (0, 0)),
            pl.BlockSpec((1, HC_), lambda i: (0, 0)),
            pl.BlockSpec((1, HC_), lambda i: (0, 0)),
            pl.BlockSpec((1, HC_), lambda i: (0, 0)),
            pl.BlockSpec((HC_, HC_), lambda i: (0, 0)),
            pl.BlockSpec((HC_, H_), lambda i: (0, 0)),
            pl.BlockSpec((HC_, H_), lambda i: (0, 0)),
        ],
        out_specs=_TABLE_SPECS,
        out_shape=_TABLE_SHAPES,
    )(pnum, pden, rrep, b, g, be, W, As_bd, Ad_bd)


def _dense_stage_c(pnum, pden, batch3, rrep, b, g, be, l1w, l1b, gf, bf,
                   l2w, l2b):
    return pl.pallas_call(
        _stage_c_body,
        grid=(NBLK_,),
        in_specs=[
            pl.BlockSpec((2, BLK_, HC_), lambda i: (0, i, 0)),
            pl.BlockSpec((2, BLK_, 2 * H_), lambda i: (0, i, 0)),
            pl.BlockSpec((1, 1, BLK_), lambda i: (i, 0, 0)),
            pl.BlockSpec((H_, HC_), lambda i: (0, 0)),
            pl.BlockSpec((1, HC_), lambda i: (0, 0)),
            pl.BlockSpec((1, HC_), lambda i: (0, 0)),
            pl.BlockSpec((1, HC_), lambda i: (0, 0)),
            pl.BlockSpec((HC_, C_), lambda i: (0, 0)),
            pl.BlockSpec((1, C_), lambda i: (0, 0)),
            pl.BlockSpec((1, C_), lambda i: (0, 0)),
            pl.BlockSpec((1, C_), lambda i: (0, 0)),
            pl.BlockSpec((C_, C_), lambda i: (0, 0)),
            pl.BlockSpec((1, C_), lambda i: (0, 0)),
        ],
        out_specs=pl.BlockSpec((G_, C_), lambda i: (0, 0)),
        out_shape=jax.ShapeDtypeStruct((G_, C_), jnp.float32),
        scratch_shapes=[
            pltpu.VMEM((G_, HC_), jnp.float32),
            pltpu.VMEM((G_, HC_), jnp.float32),
        ],
    )(pnum, pden, batch3, rrep, b, g, be, l1w, l1b, gf, bf, l2w, l2b)


def _edge_kernel_body(h_hbm, als_hbm, ald_hbm, ei_hbm, z_hbm,
                      num_hbm, den_hbm,
                      src0, src1, src2, src3, dst0, dst1, dst2, dst3,
                      sd0, sd1,
                      rows0, rows1, rows2, rows3,
                      as0, as1, as2, as3, ad0, ad1, ad2, ad3,
                      msg0, msg1, acc,
                      sis0, sis1, sis2, sis3, sid0, sid1, sid2, sid3,
                      sgr0, sgr1, sgr2, sgr3, sga0, sga1, sga2, sga3,
                      sgb0, sgb1, sgb2, sgb3, ssc0, ssc1):
    c = lax.axis_index("c")
    s = lax.axis_index("s")
    wid = c * 16 + s
    base = wid * EW_

    SRC = (src0, src1, src2, src3)
    DST = (dst0, dst1, dst2, dst3)
    SD = (sd0, sd1)
    ROWS = (rows0, rows1, rows2, rows3)
    ALS = (as0, as1, as2, as3)
    ALD = (ad0, ad1, ad2, ad3)
    MSG = (msg0, msg1)
    SIS = (sis0, sis1, sis2, sis3)
    SID = (sid0, sid1, sid2, sid3)
    SGR = (sgr0, sgr1, sgr2, sgr3)
    SGA = (sga0, sga1, sga2, sga3)
    SGB = (sgb0, sgb1, sgb2, sgb3)
    SSC = (ssc0, ssc1)

    pltpu.sync_copy(z_hbm.at[pl.ds(s * ROWS_T_, ROWS_T_)],
                    acc.at[pl.ds(s * ROWS_T_, ROWS_T_)])
    pltpu.sync_copy(z_hbm.at[pl.ds(0, KCH_)], msg0)
    pltpu.sync_copy(z_hbm.at[pl.ds(0, KCH_)], msg1)
    plsc.subcore_barrier()

    def issue_ids(it, b):
        off = base + it * KCH_
        pltpu.async_copy(ei_hbm.at[pl.ds(off, KCH_)], SRC[b], SIS[b])
        pltpu.async_copy(ei_hbm.at[pl.ds(E_ + off, KCH_)], DST[b], SID[b])

    def wait_ids(b):
        pltpu.make_async_copy(ei_hbm.at[pl.ds(0, KCH_)], SRC[b], SIS[b]).wait()
        pltpu.make_async_copy(ei_hbm.at[pl.ds(0, KCH_)], DST[b], SID[b]).wait()

    def issue_gathers(b):
        pltpu.async_copy(h_hbm.at[SRC[b]], ROWS[b], SGR[b])
        pltpu.async_copy(als_hbm.at[SRC[b]], ALS[b], SGA[b])
        pltpu.async_copy(ald_hbm.at[DST[b]], ALD[b], SGB[b])

    def wait_gathers(b):
        pltpu.make_async_copy(h_hbm.at[SRC[b]], ROWS[b], SGR[b]).wait()
        pltpu.make_async_copy(als_hbm.at[SRC[b]], ALS[b], SGA[b]).wait()
        pltpu.make_async_copy(ald_hbm.at[DST[b]], ALD[b], SGB[b]).wait()

    def issue_scatter(m):
        pltpu.async_copy(MSG[m], acc.at[SD[m]], SSC[m], add=True)

    def wait_scatter(m):
        pltpu.make_async_copy(MSG[m], acc.at[SD[m]], SSC[m]).wait()

    def compute(b, m):
        rows_v, als_v, ald_v, msg_v = ROWS[b], ALS[b], ALD[b], MSG[m]

        @plsc.parallel_loop(0, KCH_, unroll=2)
        def edge(i):
            e = als_v[i, :] + ald_v[i, :]
            e = jnp.maximum(e, 0.2 * e)
            ex = jnp.exp(e)
            msg_v[i, pl.ds(HC_, 16)] = ex
            for h in range(H_):
                bc = ex[h]
                msg_v[i, pl.ds(h * 16, 16)] = (
                    rows_v[i, pl.ds(h * 16, 16)] * bc)

    _strips = list(range(0, KCH_ - 16, 16)) + [KCH_ - 16]

    def copy_dst(b, m):
        for j in _strips:
            SD[m][pl.ds(j, 16)] = DST[b][pl.ds(j, 16)]

    def zero_sd(m):
        for j in _strips:
            SD[m][pl.ds(j, 16)] = jnp.zeros((16,), jnp.int32)

    zero_sd(0)
    zero_sd(1)
    issue_scatter(0)
    issue_scatter(1)
    for b in range(4):
        issue_ids(b, b)
    wait_ids(0)
    issue_gathers(0)
    wait_ids(1)
    issue_gathers(1)

    def quarter(it, b, m):
        wait_gathers(b)
        wait_scatter(m)
        copy_dst(b, m)

        @pl.when(it + 2 < NCH_)
        def _():
            wait_ids((b + 2) % 4)
            issue_gathers((b + 2) % 4)

        @pl.when(it + 4 < NCH_)
        def _():
            issue_ids(it + 4, b)

        compute(b, m)
        issue_scatter(m)

    def round_(t, carry):
        for q in range(4):
            it = 4 * t + q

            @pl.when(it < NCH_)
            def _():
                quarter(it, q, q % 2)

        return carry

    lax.fori_loop(0, (NCH_ + 3) // 4, round_, 0, unroll=False)
    wait_scatter(0)
    wait_scatter(1)
    plsc.subcore_barrier()
    pltpu.sync_copy(acc.at[pl.ds(s * ROWS_T_, ROWS_T_), pl.ds(0, HC_)],
                    num_hbm.at[c, pl.ds(s * ROWS_T_, ROWS_T_)])
    pltpu.sync_copy(acc.at[pl.ds(s * ROWS_T_, ROWS_T_), pl.ds(HC_, 2 * H_)],
                    den_hbm.at[c, pl.ds(s * ROWS_T_, ROWS_T_)])


def _edge_phase(h_tab, als_tab, ald_tab, edge_index, zeros_tab):
    mesh = plsc.VectorSubcoreMesh(core_axis_name="c", subcore_axis_name="s")
    fn = functools.partial(
        pl.kernel,
        mesh=mesh,
        compiler_params=pltpu.CompilerParams(use_tc_tiling_on_sc=False),
        out_type=(
            jax.ShapeDtypeStruct((2, NPAD_, HC_), jnp.float32),
            jax.ShapeDtypeStruct((2, NPAD_, 2 * H_), jnp.float32),
        ),
        scratch_types=(
            [pltpu.VMEM((KCH_,), jnp.int32)] * 10 +
            [pltpu.VMEM((KCH_, HC_), jnp.float32)] * 4 +
            [pltpu.VMEM((KCH_, 2 * H_), jnp.float32)] * 8 +
            [pltpu.VMEM((KCH_, TW_), jnp.float32)] * 2 +
            [pltpu.VMEM_SHARED((NPAD_, TW_), jnp.float32)] +
            [pltpu.SemaphoreType.DMA] * 22
        ),
    )(_edge_kernel_body)
    return fn(h_tab, als_tab, ald_tab, edge_index, zeros_tab)


def _block_diag_attn(a):
    return (a[:, :, None] * jnp.eye(H_, dtype=jnp.float32)[:, None, :]
            ).reshape(H_ * C_, H_)


def kernel(x, edge_index, edge_attr, batch, W0, a_src0, a_dst0, b0, g0, be0,
           W1, a_src1, a_dst1, b1, g1, be1, lin1_W, lin1_b, gf, bf,
           lin2_W, lin2_b):
    x2d = jnp.squeeze(x.astype(jnp.float32), axis=1)
    ei = edge_index.astype(jnp.int32).reshape(2 * E_)
    batch3 = batch.astype(jnp.int32).reshape(NBLK_, 1, BLK_)

    As0 = _block_diag_attn(a_src0)
    Ad0 = _block_diag_attn(a_dst0)
    As1 = _block_diag_attn(a_src1)
    Ad1 = _block_diag_attn(a_dst1)
    rrep = (jnp.arange(HC_)[None, :] // C_ ==
            jnp.arange(H_)[:, None]).astype(jnp.float32)
    zeros_tab = jnp.zeros((NPAD_, TW_), jnp.float32)

    r2 = lambda t: t.reshape(1, -1).astype(jnp.float32)

    h0, als0, ald0 = _dense_stage_a(x2d, W0, As0, Ad0)
    num0, den0 = _edge_phase(h0, als0, ald0, ei, zeros_tab)
    h1, als1, ald1 = _dense_stage_b(num0, den0, rrep, r2(b0), r2(g0),
                                    r2(be0), W1, As1, Ad1)
    num1, den1 = _edge_phase(h1, als1, ald1, ei, zeros_tab)
    return _dense_stage_c(num1, den1, batch3, rrep, r2(b1), r2(g1), r2(be1),
                          lin1_W, r2(lin1_b), r2(gf), r2(bf),
                          lin2_W, r2(lin2_b))
